# Initial kernel scaffold; baseline (speedup 1.0000x reference)
import jax
import jax.numpy as jnp
from jax.experimental import pallas as pl


def kernel(x, pos_embed):
    raise NotImplementedError("write your pallas kernel here")


# speedup vs baseline: 1.4747x; 1.4747x over previous
import jax
import jax.numpy as jnp
from jax.experimental import pallas as pl


def _add_pe_kernel(x_ref, pe_ref, o_ref):
    o_ref[...] = x_ref[...] + pe_ref[...]


def kernel(x, pos_embed):
    B, T, D = x.shape
    pe = pos_embed[:T]

    SBLK = 256
    grid = (T // SBLK, B)

    out = pl.pallas_call(
        _add_pe_kernel,
        grid=grid,
        in_specs=[
            pl.BlockSpec((1, SBLK, D), lambda s, b: (b, s, 0)),
            pl.BlockSpec((SBLK, D), lambda s, b: (s, 0)),
        ],
        out_specs=pl.BlockSpec((1, SBLK, D), lambda s, b: (b, s, 0)),
        out_shape=jax.ShapeDtypeStruct((B, T, D), x.dtype),
    )(x, pe)
    return out


# speedup vs baseline: 1.9400x; 1.3155x over previous
import jax
import jax.numpy as jnp
from jax.experimental import pallas as pl


def _add_pe_kernel(x_ref, pe_ref, o_ref):
    o_ref[...] = x_ref[...] + pe_ref[...]


def kernel(x, pos_embed):
    B, T, D = x.shape
    pe = pos_embed[:T]

    SBLK = 512
    grid = (T // SBLK, B)

    out = pl.pallas_call(
        _add_pe_kernel,
        grid=grid,
        in_specs=[
            pl.BlockSpec((1, SBLK, D), lambda s, b: (b, s, 0)),
            pl.BlockSpec((SBLK, D), lambda s, b: (s, 0)),
        ],
        out_specs=pl.BlockSpec((1, SBLK, D), lambda s, b: (b, s, 0)),
        out_shape=jax.ShapeDtypeStruct((B, T, D), x.dtype),
    )(x, pe)
    return out


# speedup vs baseline: 2.1047x; 1.0849x over previous
import jax
import jax.numpy as jnp
from jax.experimental import pallas as pl


def _add_pe_kernel(x_ref, pe_ref, o_ref):
    o_ref[...] = x_ref[...] + pe_ref[...]


def kernel(x, pos_embed):
    B, T, D = x.shape
    pe = pos_embed[:T]

    SBLK = 1024
    grid = (T // SBLK, B)

    out = pl.pallas_call(
        _add_pe_kernel,
        grid=grid,
        in_specs=[
            pl.BlockSpec((1, SBLK, D), lambda s, b: (b, s, 0)),
            pl.BlockSpec((SBLK, D), lambda s, b: (s, 0)),
        ],
        out_specs=pl.BlockSpec((1, SBLK, D), lambda s, b: (b, s, 0)),
        out_shape=jax.ShapeDtypeStruct((B, T, D), x.dtype),
    )(x, pe)
    return out


# speedup vs baseline: 2.3013x; 1.0934x over previous
import jax
import jax.numpy as jnp
from jax.experimental import pallas as pl


def _add_pe_kernel(x_ref, pe_ref, o_ref):
    o_ref[...] = x_ref[...] + pe_ref[...]


def kernel(x, pos_embed):
    B, T, D = x.shape
    pe = pos_embed[:T]

    SBLK = 2048
    grid = (T // SBLK, B)

    out = pl.pallas_call(
        _add_pe_kernel,
        grid=grid,
        in_specs=[
            pl.BlockSpec((1, SBLK, D), lambda s, b: (b, s, 0)),
            pl.BlockSpec((SBLK, D), lambda s, b: (s, 0)),
        ],
        out_specs=pl.BlockSpec((1, SBLK, D), lambda s, b: (b, s, 0)),
        out_shape=jax.ShapeDtypeStruct((B, T, D), x.dtype),
    )(x, pe)
    return out
